# baseline (speedup 1.0000x reference)
import jax
import jax.numpy as jnp
import numpy as np
from jax.experimental import pallas as pl
from jax.experimental.pallas import tpu as pltpu

_NF = 12
_E = 22
_P = 26
_FEAT = 2 * _NF + _E
_COLS = _P * _FEAT
_ROWS_PER_BLOCK = 512


def _body(x_ref, sel_ref, f_ref, ph_ref, embrow_ref, out_ref, mask_ref):
    x = x_ref[...]
    ok = jnp.isfinite(x)
    xc = jnp.where(ok, x, 0.0)
    sel = sel_ref[...]
    xg = jnp.dot(xc, sel, preferred_element_type=jnp.float32,
                 precision=jax.lax.Precision.HIGHEST)
    okg = jnp.dot(ok.astype(jnp.float32), sel, preferred_element_type=jnp.float32)
    ang = xg * f_ref[...] + ph_ref[...]
    s = jnp.sin(ang) * okg
    s = jnp.where(jnp.isnan(s), 0.0, s)
    out_ref[...] = s + embrow_ref[...]
    mask_ref[...] = jnp.isnan(x)[:, None, :]


def kernel(predictor_values, var_indices, emb_table):
    b, l, p = predictor_values.shape
    e = emb_table.shape[1]
    nf = _NF
    feat = 2 * nf + e
    cols = p * feat
    rows = b * l

    x2d = predictor_values.reshape(rows, p)

    freqs = np.pi * (2.0 ** np.arange(nf, dtype=np.float32))
    f_row = np.zeros((p, feat), dtype=np.float32)
    f_row[:, :nf] = freqs
    f_row[:, nf:2 * nf] = freqs
    ph_row = np.zeros((p, feat), dtype=np.float32)
    ph_row[:, nf:2 * nf] = np.float32(np.pi / 2)
    f_row = jnp.asarray(f_row.reshape(1, cols))
    ph_row = jnp.asarray(ph_row.reshape(1, cols))

    sel_np = np.zeros((p, p, feat), dtype=np.float32)
    for q in range(p):
        sel_np[q, q, :] = 1.0
    sel = jnp.asarray(sel_np.reshape(p, cols))

    emb_g = jnp.take(emb_table, var_indices, axis=0)
    embrow = jnp.zeros((p, feat), dtype=jnp.float32).at[:, 2 * nf:].set(emb_g)
    embrow = embrow.reshape(1, cols)

    r = _ROWS_PER_BLOCK
    grid = (rows // r,)

    out2d, mask = pl.pallas_call(
        _body,
        grid=grid,
        in_specs=[
            pl.BlockSpec((r, p), lambda i: (i, 0)),
            pl.BlockSpec((p, cols), lambda i: (0, 0)),
            pl.BlockSpec((1, cols), lambda i: (0, 0)),
            pl.BlockSpec((1, cols), lambda i: (0, 0)),
            pl.BlockSpec((1, cols), lambda i: (0, 0)),
        ],
        out_specs=[
            pl.BlockSpec((r, cols), lambda i: (i, 0)),
            pl.BlockSpec((r, 1, p), lambda i: (i, 0, 0)),
        ],
        out_shape=[
            jax.ShapeDtypeStruct((rows, cols), jnp.float32),
            jax.ShapeDtypeStruct((rows, 1, p), jnp.bool_),
        ],
        compiler_params=pltpu.CompilerParams(
            dimension_semantics=("arbitrary",),
        ),
    )(x2d, sel, f_row, ph_row, embrow)

    return out2d.reshape(rows, p, feat), mask


# speedup vs baseline: 1.8560x; 1.8560x over previous
import jax
import jax.numpy as jnp
import numpy as np
from jax.experimental import pallas as pl
from jax.experimental.pallas import tpu as pltpu

_NF = 12
_E = 22
_P = 26
_FEAT = 2 * _NF + _E
_COLS = _P * _FEAT
_ROWS_PER_BLOCK = 512


_SINPI_C = (-3.141592117449171, 2.026090489231321, -0.5237850942341205,
            0.0744342469476893, -0.005945976187534801)


def _sinpi(m):
    n = jax.lax.round(m * 0.5, jax.lax.RoundingMethod.TO_NEAREST_EVEN)
    r = m - 2.0 * n
    u = r * r
    p = _SINPI_C[4]
    p = p * u + _SINPI_C[3]
    p = p * u + _SINPI_C[2]
    p = p * u + _SINPI_C[1]
    p = p * u + _SINPI_C[0]
    return r * (u - 1.0) * p


def _body(x_ref, sel_ref, f_ref, ph_ref, embrow_ref, out_ref, mask_ref):
    x = x_ref[...]
    ok = jnp.isfinite(x)
    xc = jnp.where(ok, x, 0.0)
    sel = sel_ref[...]
    xg = jnp.dot(xc, sel, preferred_element_type=jnp.float32,
                 precision=jax.lax.Precision.HIGHEST)
    okg = jnp.dot(ok.astype(jnp.float32), sel, preferred_element_type=jnp.float32)
    m = xg * f_ref[...] + ph_ref[...]
    s = _sinpi(m) * okg
    s = jnp.where(jnp.isnan(s), 0.0, s)
    out_ref[...] = s + embrow_ref[...]
    mask_ref[...] = jnp.isnan(x)[:, None, :]


def kernel(predictor_values, var_indices, emb_table):
    b, l, p = predictor_values.shape
    e = emb_table.shape[1]
    nf = _NF
    feat = 2 * nf + e
    cols = p * feat
    rows = b * l

    x2d = predictor_values.reshape(rows, p)

    freqs = 2.0 ** np.arange(nf, dtype=np.float32)
    f_row = np.zeros((p, feat), dtype=np.float32)
    f_row[:, :nf] = freqs
    f_row[:, nf:2 * nf] = freqs
    ph_row = np.zeros((p, feat), dtype=np.float32)
    ph_row[:, nf:2 * nf] = np.float32(0.5)
    f_row = jnp.asarray(f_row.reshape(1, cols))
    ph_row = jnp.asarray(ph_row.reshape(1, cols))

    sel_np = np.zeros((p, p, feat), dtype=np.float32)
    for q in range(p):
        sel_np[q, q, :] = 1.0
    sel = jnp.asarray(sel_np.reshape(p, cols))

    emb_g = jnp.take(emb_table, var_indices, axis=0)
    embrow = jnp.zeros((p, feat), dtype=jnp.float32).at[:, 2 * nf:].set(emb_g)
    embrow = embrow.reshape(1, cols)

    r = _ROWS_PER_BLOCK
    grid = (rows // r,)

    out2d, mask = pl.pallas_call(
        _body,
        grid=grid,
        in_specs=[
            pl.BlockSpec((r, p), lambda i: (i, 0)),
            pl.BlockSpec((p, cols), lambda i: (0, 0)),
            pl.BlockSpec((1, cols), lambda i: (0, 0)),
            pl.BlockSpec((1, cols), lambda i: (0, 0)),
            pl.BlockSpec((1, cols), lambda i: (0, 0)),
        ],
        out_specs=[
            pl.BlockSpec((r, cols), lambda i: (i, 0)),
            pl.BlockSpec((r, 1, p), lambda i: (i, 0, 0)),
        ],
        out_shape=[
            jax.ShapeDtypeStruct((rows, cols), jnp.float32),
            jax.ShapeDtypeStruct((rows, 1, p), jnp.bool_),
        ],
        compiler_params=pltpu.CompilerParams(
            dimension_semantics=("arbitrary",),
        ),
    )(x2d, sel, f_row, ph_row, embrow)

    return out2d.reshape(rows, p, feat), mask
